# Initial kernel scaffold; baseline (speedup 1.0000x reference)
import functools

import jax
import jax.numpy as jnp
from jax import lax
from jax.experimental import pallas as pl
from jax.experimental.pallas import tpu as pltpu
from jax.experimental.pallas import tpu_sc as plsc

_N = 10000
_H = 128
_E = 320000
_ED = 16
_NC = 2
_NS = 16
_NW = _NC * _NS
_EPW = _E // _NW
_CHUNK = 80
_NCHUNK = _EPW // _CHUNK
_RPT = _N // _NS
_BLK = 1000
_EBLK = 4000

_f32 = jnp.float32


def _full(shape):
    nd = len(shape)
    return pl.BlockSpec(shape, lambda i: (0,) * nd)


def _rows(width):
    return pl.BlockSpec((_BLK, width), lambda i: (i, 0))


def _k0_body(nf, sp, co, emb_pad, wa, wb, wc, inb, ws, wd, wcs, wcd, mb1,
             x_o, a_o, b_o):
    oh = (sp[...] == lax.broadcasted_iota(jnp.int32, (1, _H), 1)).astype(_f32)
    e = jnp.dot(oh, emb_pad[...], preferred_element_type=_f32)
    x = (jnp.dot(nf[...], wa[...], preferred_element_type=_f32)
         + jnp.dot(e, wb[...], preferred_element_type=_f32)
         + co[...] * wc[...] + inb[...])
    x_o[...] = x
    a_o[...] = jnp.dot(x, ws[...], preferred_element_type=_f32) + co[...] * wcs[...]
    b_o[...] = (jnp.dot(x, wd[...], preferred_element_type=_f32)
                + co[...] * wcd[...] + mb1[...])


def _k0(nf, sp, co, emb_pad, wa, wb, wc, inb, ws, wd, wcs, wcd, mb1):
    return pl.pallas_call(
        _k0_body,
        grid=(_N // _BLK,),
        in_specs=[_rows(_H), _rows(1), _rows(1), _full((_H, _H)),
                  _full((_H, _H)), _full((_H, _H)), _full((1, _H)),
                  _full((1, _H)), _full((_H, _H)), _full((_H, _H)),
                  _full((1, _H)), _full((1, _H)), _full((1, _H))],
        out_specs=[_rows(_H), _rows(_H), _rows(_H)],
        out_shape=[jax.ShapeDtypeStruct((_N, _H), _f32)] * 3,
    )(nf, sp, co, emb_pad, wa, wb, wc, inb, ws, wd, wcs, wcd, mb1)


def _kc_body(ef, we0, we1, c0_o, c1_o):
    c0_o[...] = jnp.dot(ef[...], we0[...], preferred_element_type=_f32)
    c1_o[...] = jnp.dot(ef[...], we1[...], preferred_element_type=_f32)


def _kc(ef, we0, we1):
    return pl.pallas_call(
        _kc_body,
        grid=(_E // _EBLK,),
        in_specs=[pl.BlockSpec((_EBLK, _ED), lambda i: (i, 0)),
                  _full((_ED, _H)), _full((_ED, _H))],
        out_specs=[pl.BlockSpec((_EBLK, _H), lambda i: (i, 0))] * 2,
        out_shape=[jax.ShapeDtypeStruct((_E, _H), _f32)] * 2,
    )(ef, we0, we1)


def _sc_edge_body(a_hbm, b_hbm, c_hbm, src_hbm, dst_hbm, z128, z16, ones_h,
                  s_out, d_out,
                  srcv, dstv, rows_a, rows_b, rows_c, onesv,
                  sem_a, sem_b, sem_c, s_sh, d_sh):
    c = lax.axis_index("c")
    s = lax.axis_index("s")
    wid = s * _NC + c
    pltpu.sync_copy(z128, s_sh.at[pl.ds(s * _RPT, _RPT)])
    pltpu.sync_copy(z16, d_sh.at[pl.ds(s * _RPT, _RPT)])
    pltpu.sync_copy(ones_h, onesv)
    plsc.subcore_barrier()

    base0 = wid * _EPW

    def chunk_body(i, carry):
        base = base0 + i * _CHUNK
        pltpu.sync_copy(src_hbm.at[pl.ds(base, _CHUNK)], srcv)
        pltpu.sync_copy(dst_hbm.at[pl.ds(base, _CHUNK)], dstv)
        ca = pltpu.async_copy(a_hbm.at[srcv], rows_a, sem_a)
        cb = pltpu.async_copy(b_hbm.at[dstv], rows_b, sem_b)
        cc = pltpu.async_copy(c_hbm.at[pl.ds(base, _CHUNK)], rows_c, sem_c)
        ca.wait()
        cb.wait()
        cc.wait()

        def row_body(r, carry2):
            for k in range(_H // 16):
                sl = pl.ds(k * 16, 16)
                t = rows_a[r, sl] + rows_b[r, sl] + rows_c[r, sl]
                rows_c[r, sl] = t / (1.0 + jnp.exp(-t))
            return carry2

        lax.fori_loop(0, _CHUNK, row_body, 0)
        pltpu.sync_copy(rows_c, s_sh.at[dstv], add=True)
        pltpu.sync_copy(onesv, d_sh.at[dstv], add=True)
        return carry

    lax.fori_loop(0, _NCHUNK, chunk_body, 0)
    plsc.subcore_barrier()
    pltpu.sync_copy(s_sh.at[pl.ds(s * _RPT, _RPT)],
                    s_out.at[c, pl.ds(s * _RPT, _RPT)])
    pltpu.sync_copy(d_sh.at[pl.ds(s * _RPT, _RPT)],
                    d_out.at[c, pl.ds(s * _RPT, _RPT)])


_sc_edge = functools.partial(
    pl.kernel,
    mesh=plsc.VectorSubcoreMesh(core_axis_name="c", subcore_axis_name="s",
                                num_cores=_NC, num_subcores=_NS),
    out_type=[jax.ShapeDtypeStruct((_NC, _N, _H), _f32),
              jax.ShapeDtypeStruct((_NC, _N, 16), _f32)],
    scratch_types=[
        pltpu.VMEM((_CHUNK,), jnp.int32),
        pltpu.VMEM((_CHUNK,), jnp.int32),
        pltpu.VMEM((_CHUNK, _H), _f32),
        pltpu.VMEM((_CHUNK, _H), _f32),
        pltpu.VMEM((_CHUNK, _H), _f32),
        pltpu.VMEM((_CHUNK, 16), _f32),
        pltpu.SemaphoreType.DMA,
        pltpu.SemaphoreType.DMA,
        pltpu.SemaphoreType.DMA,
        pltpu.VMEM_SHARED((_N, _H), _f32),
        pltpu.VMEM_SHARED((_N, 16), _f32),
    ],
)(_sc_edge_body)


def _post_common(x, s0, s1, d0, d1, co, mw2, mb2, ux, ua, uc, ub1, uw2, ub2,
                 g, b):
    agg = (jnp.dot(s0[...] + s1[...], mw2[...], preferred_element_type=_f32)
           + (d0[...] + d1[...]) * mb2[...])
    t = (jnp.dot(x[...], ux[...], preferred_element_type=_f32)
         + jnp.dot(agg, ua[...], preferred_element_type=_f32)
         + co[...] * uc[...] + ub1[...])
    h = t * jax.nn.sigmoid(t)
    y = x[...] + jnp.dot(h, uw2[...], preferred_element_type=_f32) + ub2[...]
    mu = jnp.mean(y, axis=-1, keepdims=True)
    yc = y - mu
    var = jnp.mean(yc * yc, axis=-1, keepdims=True)
    return yc * lax.rsqrt(var + 1e-5) * g[...] + b[...]


def _kp_body(x, s0, s1, d0, d1, co, mw2, mb2, ux, ua, uc, ub1, uw2, ub2, g, b,
             ws, wd, wcs, wcd, mb1, x_o, a_o, b_o):
    xn = _post_common(x, s0, s1, d0, d1, co, mw2, mb2, ux, ua, uc, ub1, uw2,
                      ub2, g, b)
    x_o[...] = xn
    a_o[...] = jnp.dot(xn, ws[...], preferred_element_type=_f32) + co[...] * wcs[...]
    b_o[...] = (jnp.dot(xn, wd[...], preferred_element_type=_f32)
                + co[...] * wcd[...] + mb1[...])


def _kp(x, s0, s1, d0, d1, co, mw2, mb2, ux, ua, uc, ub1, uw2, ub2, g, b,
        ws, wd, wcs, wcd, mb1):
    return pl.pallas_call(
        _kp_body,
        grid=(_N // _BLK,),
        in_specs=[_rows(_H), _rows(_H), _rows(_H), _rows(1), _rows(1),
                  _rows(1), _full((_H, _H)), _full((1, _H)), _full((_H, _H)),
                  _full((_H, _H)), _full((1, _H)), _full((1, _H)),
                  _full((_H, _H)), _full((1, _H)), _full((1, _H)),
                  _full((1, _H)), _full((_H, _H)), _full((_H, _H)),
                  _full((1, _H)), _full((1, _H)), _full((1, _H))],
        out_specs=[_rows(_H), _rows(_H), _rows(_H)],
        out_shape=[jax.ShapeDtypeStruct((_N, _H), _f32)] * 3,
    )(x, s0, s1, d0, d1, co, mw2, mb2, ux, ua, uc, ub1, uw2, ub2, g, b,
      ws, wd, wcs, wcd, mb1)


def _kr_body(x, s0, s1, d0, d1, co, mw2, mb2, ux, ua, uc, ub1, uw2, ub2, g, b,
             rw1, rb1, rw2, rb2, o_o):
    xn = _post_common(x, s0, s1, d0, d1, co, mw2, mb2, ux, ua, uc, ub1, uw2,
                      ub2, g, b)
    t2 = jnp.dot(xn, rw1[...], preferred_element_type=_f32) + rb1[...]
    h2 = t2 * jax.nn.sigmoid(t2)
    o_o[...] = jnp.dot(h2, rw2[...], preferred_element_type=_f32) + rb2[...]


def _kr(x, s0, s1, d0, d1, co, mw2, mb2, ux, ua, uc, ub1, uw2, ub2, g, b,
        rw1, rb1, rw2, rb2):
    return pl.pallas_call(
        _kr_body,
        grid=(_N // _BLK,),
        in_specs=[_rows(_H), _rows(_H), _rows(_H), _rows(1), _rows(1),
                  _rows(1), _full((_H, _H)), _full((1, _H)), _full((_H, _H)),
                  _full((_H, _H)), _full((1, _H)), _full((1, _H)),
                  _full((_H, _H)), _full((1, _H)), _full((1, _H)),
                  _full((1, _H)), _full((_H, _H)), _full((1, _H)),
                  _full((_H, 1)), _full((1, 1))],
        out_specs=[_rows(1)],
        out_shape=[jax.ShapeDtypeStruct((_N, 1), _f32)],
    )(x, s0, s1, d0, d1, co, mw2, mb2, ux, ua, uc, ub1, uw2, ub2, g, b,
      rw1, rb1, rw2, rb2)[0]


def _split_layer(L):
    mw1 = L["mw1"]
    return dict(
        ws=mw1[:_H], wd=mw1[_H:2 * _H], we=mw1[2 * _H:2 * _H + _ED],
        wcs=mw1[2 * _H + _ED:2 * _H + _ED + 1],
        wcd=mw1[2 * _H + _ED + 1:2 * _H + _ED + 2],
        mb1=L["mb1"].reshape(1, _H), mw2=L["mw2"],
        mb2=L["mb2"].reshape(1, _H),
        ux=L["uw1"][:_H], ua=L["uw1"][_H:2 * _H],
        uc=L["uw1"][2 * _H:2 * _H + 1],
        ub1=L["ub1"].reshape(1, _H), uw2=L["uw2"],
        ub2=L["ub2"].reshape(1, _H),
        g=L["ln_g"].reshape(1, _H), b=L["ln_b"].reshape(1, _H),
    )


def kernel(node_features, species, edge_index, edge_features, coordination,
           params):
    p = params
    src = edge_index[0].astype(jnp.int32)
    dst = edge_index[1].astype(jnp.int32)
    sp2 = species.reshape(_N, 1).astype(jnp.int32)
    co2 = coordination.reshape(_N, 1).astype(_f32)
    emb_pad = jnp.zeros((_H, _H), _f32).at[:p["emb"].shape[0]].set(p["emb"])
    in_w = p["in_w"]
    wa, wb, wc = in_w[:_H], in_w[_H:2 * _H], in_w[2 * _H:2 * _H + 1]
    inb = p["in_b"].reshape(1, _H)
    L0 = _split_layer(p["layers"][0])
    L1 = _split_layer(p["layers"][1])

    z128 = jnp.zeros((_RPT, _H), _f32)
    z16 = jnp.zeros((_RPT, 16), _f32)
    ones_h = jnp.ones((_CHUNK, 16), _f32)

    x, a, b = _k0(node_features, sp2, co2, emb_pad, wa, wb, wc, inb,
                  L0["ws"], L0["wd"], L0["wcs"], L0["wcd"], L0["mb1"])
    c0, c1 = _kc(edge_features, L0["we"], L1["we"])

    s, d = _sc_edge(a, b, c0, src, dst, z128, z16, ones_h)
    x, a, b = _kp(x, s[0], s[1], d[0, :, :1], d[1, :, :1], co2,
                  L0["mw2"], L0["mb2"], L0["ux"], L0["ua"], L0["uc"],
                  L0["ub1"], L0["uw2"], L0["ub2"], L0["g"], L0["b"],
                  L1["ws"], L1["wd"], L1["wcs"], L1["wcd"], L1["mb1"])

    s, d = _sc_edge(a, b, c1, src, dst, z128, z16, ones_h)
    out = _kr(x, s[0], s[1], d[0, :, :1], d[1, :, :1], co2,
              L1["mw2"], L1["mb2"], L1["ux"], L1["ua"], L1["uc"],
              L1["ub1"], L1["uw2"], L1["ub2"], L1["g"], L1["b"],
              p["rw1"], p["rb1"].reshape(1, _H), p["rw2"],
              p["rb2"].reshape(1, 1))
    return out.reshape(_N)


# speedup vs baseline: 4.1277x; 4.1277x over previous
import functools

import jax
import jax.numpy as jnp
from jax import lax
from jax.experimental import pallas as pl
from jax.experimental.pallas import tpu as pltpu
from jax.experimental.pallas import tpu_sc as plsc

_N = 10000
_H = 128
_E = 320000
_ED = 16
_NC = 2
_NS = 16
_NW = _NC * _NS
_EPW = _E // _NW
_CHUNK = 40
_NCHUNK = _EPW // _CHUNK
_NP = 10240
_RPT = _NP // _NS
_BLK = 1000
_EBLK = 4000

_f32 = jnp.float32


def _dot(a, b):
    return jnp.dot(a, b, preferred_element_type=_f32,
                   precision=lax.Precision.HIGHEST)


def _full(shape):
    nd = len(shape)
    return pl.BlockSpec(shape, lambda i: (0,) * nd)


def _rows(width):
    return pl.BlockSpec((_BLK, width), lambda i: (i, 0))


def _k0_body(nf, sp, co, emb_pad, wa, wb, wc, inb, ws, wd, wcs, wcd, mb1,
             x_o, a_o, b_o):
    oh = (sp[...] == lax.broadcasted_iota(jnp.int32, (1, _H), 1)).astype(_f32)
    e = _dot(oh, emb_pad[...])
    x = (_dot(nf[...], wa[...])
         + _dot(e, wb[...])
         + co[...] * wc[...] + inb[...])
    x_o[...] = x
    a_o[...] = _dot(x, ws[...]) + co[...] * wcs[...]
    b_o[...] = (_dot(x, wd[...])
                + co[...] * wcd[...] + mb1[...])


def _k0(nf, sp, co, emb_pad, wa, wb, wc, inb, ws, wd, wcs, wcd, mb1):
    return pl.pallas_call(
        _k0_body,
        grid=(_N // _BLK,),
        in_specs=[_rows(_H), _rows(1), _rows(1), _full((_H, _H)),
                  _full((_H, _H)), _full((_H, _H)), _full((1, _H)),
                  _full((1, _H)), _full((_H, _H)), _full((_H, _H)),
                  _full((1, _H)), _full((1, _H)), _full((1, _H))],
        out_specs=[_rows(_H), _rows(_H), _rows(_H)],
        out_shape=[jax.ShapeDtypeStruct((_N, _H), _f32)] * 3,
    )(nf, sp, co, emb_pad, wa, wb, wc, inb, ws, wd, wcs, wcd, mb1)


def _kc_body(ef, we0, we1, c0_o, c1_o):
    c0_o[...] = _dot(ef[...], we0[...])
    c1_o[...] = _dot(ef[...], we1[...])


def _kc(ef, we0, we1):
    return pl.pallas_call(
        _kc_body,
        grid=(_E // _EBLK,),
        in_specs=[pl.BlockSpec((_EBLK, _ED), lambda i: (i, 0)),
                  _full((_ED, _H)), _full((_ED, _H))],
        out_specs=[pl.BlockSpec((_EBLK, _H), lambda i: (i, 0))] * 2,
        out_shape=[jax.ShapeDtypeStruct((_E, _H), _f32)] * 2,
    )(ef, we0, we1)


def _sc_edge_body(a_hbm, b_hbm, c_hbm, src_hbm, dst_hbm, z128,
                  s_out,
                  srcv, dstv, rows_a, rows_b, rows_c,
                  sem_a, sem_b, sem_c, s_sh):
    c = lax.axis_index("c")
    s = lax.axis_index("s")
    wid = s * _NC + c
    pltpu.sync_copy(z128, s_sh.at[pl.ds(s * _RPT, _RPT)])
    plsc.subcore_barrier()

    base0 = wid * _EPW

    def chunk_body(i, carry):
        base = base0 + i * _CHUNK
        pltpu.sync_copy(src_hbm.at[pl.ds(base, _CHUNK)], srcv)
        pltpu.sync_copy(dst_hbm.at[pl.ds(base, _CHUNK)], dstv)
        ca = pltpu.async_copy(a_hbm.at[srcv], rows_a, sem_a)
        cb = pltpu.async_copy(b_hbm.at[dstv], rows_b, sem_b)
        cc = pltpu.async_copy(c_hbm.at[pl.ds(base, _CHUNK)], rows_c, sem_c)
        ca.wait()
        cb.wait()
        cc.wait()

        def row_body(r, carry2):
            for k in range(_H // 16):
                sl = pl.ds(k * 16, 16)
                t = rows_a[r, sl] + rows_b[r, sl] + rows_c[r, sl]
                rows_c[r, sl] = t / (1.0 + jnp.exp(-t))
            return carry2

        lax.fori_loop(0, _CHUNK, row_body, 0)
        pltpu.sync_copy(rows_c, s_sh.at[dstv], add=True)
        return carry

    lax.fori_loop(0, _NCHUNK, chunk_body, 0)
    plsc.subcore_barrier()
    pltpu.sync_copy(s_sh.at[pl.ds(s * _RPT, _RPT)],
                    s_out.at[c, pl.ds(s * _RPT, _RPT)])


_sc_edge = functools.partial(
    pl.kernel,
    mesh=plsc.VectorSubcoreMesh(core_axis_name="c", subcore_axis_name="s",
                                num_cores=_NC, num_subcores=_NS),
    out_type=jax.ShapeDtypeStruct((_NC, _NP, _H), _f32),
    scratch_types=[
        pltpu.VMEM((_CHUNK,), jnp.int32),
        pltpu.VMEM((_CHUNK,), jnp.int32),
        pltpu.VMEM((_CHUNK, _H), _f32),
        pltpu.VMEM((_CHUNK, _H), _f32),
        pltpu.VMEM((_CHUNK, _H), _f32),
        pltpu.SemaphoreType.DMA,
        pltpu.SemaphoreType.DMA,
        pltpu.SemaphoreType.DMA,
        pltpu.VMEM_SHARED((_NP, _H), _f32),
    ],
)(_sc_edge_body)


def _sc_deg_body(dst_hbm, z128, ones_h, d_out, dstv, onesv, d_sh):
    c = lax.axis_index("c")
    s = lax.axis_index("s")
    wid = s * _NC + c
    pltpu.sync_copy(z128, d_sh.at[pl.ds(s * _RPT, _RPT)])
    pltpu.sync_copy(ones_h, onesv)
    plsc.subcore_barrier()

    base0 = wid * _EPW

    def chunk_body(i, carry):
        base = base0 + i * _CHUNK
        pltpu.sync_copy(dst_hbm.at[pl.ds(base, _CHUNK)], dstv)
        pltpu.sync_copy(onesv, d_sh.at[dstv], add=True)
        return carry

    lax.fori_loop(0, _NCHUNK, chunk_body, 0)
    plsc.subcore_barrier()
    pltpu.sync_copy(d_sh.at[pl.ds(s * _RPT, _RPT)],
                    d_out.at[c, pl.ds(s * _RPT, _RPT)])


_sc_deg = functools.partial(
    pl.kernel,
    mesh=plsc.VectorSubcoreMesh(core_axis_name="c", subcore_axis_name="s",
                                num_cores=_NC, num_subcores=_NS),
    out_type=jax.ShapeDtypeStruct((_NC, _NP, _H), _f32),
    scratch_types=[
        pltpu.VMEM((_CHUNK,), jnp.int32),
        pltpu.VMEM((_CHUNK, _H), _f32),
        pltpu.VMEM_SHARED((_NP, _H), _f32),
    ],
)(_sc_deg_body)


def _post_common(x, s0, s1, d0, d1, co, mw2, mb2, ux, ua, uc, ub1, uw2, ub2,
                 g, b):
    agg = (_dot(s0[...] + s1[...], mw2[...])
           + (d0[...] + d1[...]) * mb2[...])
    t = (_dot(x[...], ux[...])
         + _dot(agg, ua[...])
         + co[...] * uc[...] + ub1[...])
    h = t * jax.nn.sigmoid(t)
    y = x[...] + _dot(h, uw2[...]) + ub2[...]
    mu = jnp.mean(y, axis=-1, keepdims=True)
    yc = y - mu
    var = jnp.mean(yc * yc, axis=-1, keepdims=True)
    return yc * lax.rsqrt(var + 1e-5) * g[...] + b[...]


def _kp_body(x, s0, s1, d0, d1, co, mw2, mb2, ux, ua, uc, ub1, uw2, ub2, g, b,
             ws, wd, wcs, wcd, mb1, x_o, a_o, b_o):
    xn = _post_common(x, s0, s1, d0, d1, co, mw2, mb2, ux, ua, uc, ub1, uw2,
                      ub2, g, b)
    x_o[...] = xn
    a_o[...] = _dot(xn, ws[...]) + co[...] * wcs[...]
    b_o[...] = (_dot(xn, wd[...])
                + co[...] * wcd[...] + mb1[...])


def _kp(x, s0, s1, d0, d1, co, mw2, mb2, ux, ua, uc, ub1, uw2, ub2, g, b,
        ws, wd, wcs, wcd, mb1):
    return pl.pallas_call(
        _kp_body,
        grid=(_N // _BLK,),
        in_specs=[_rows(_H), _rows(_H), _rows(_H), _rows(1), _rows(1),
                  _rows(1), _full((_H, _H)), _full((1, _H)), _full((_H, _H)),
                  _full((_H, _H)), _full((1, _H)), _full((1, _H)),
                  _full((_H, _H)), _full((1, _H)), _full((1, _H)),
                  _full((1, _H)), _full((_H, _H)), _full((_H, _H)),
                  _full((1, _H)), _full((1, _H)), _full((1, _H))],
        out_specs=[_rows(_H), _rows(_H), _rows(_H)],
        out_shape=[jax.ShapeDtypeStruct((_N, _H), _f32)] * 3,
    )(x, s0, s1, d0, d1, co, mw2, mb2, ux, ua, uc, ub1, uw2, ub2, g, b,
      ws, wd, wcs, wcd, mb1)


def _kr_body(x, s0, s1, d0, d1, co, mw2, mb2, ux, ua, uc, ub1, uw2, ub2, g, b,
             rw1, rb1, rw2, rb2, o_o):
    xn = _post_common(x, s0, s1, d0, d1, co, mw2, mb2, ux, ua, uc, ub1, uw2,
                      ub2, g, b)
    t2 = _dot(xn, rw1[...]) + rb1[...]
    h2 = t2 * jax.nn.sigmoid(t2)
    o_o[...] = _dot(h2, rw2[...]) + rb2[...]


def _kr(x, s0, s1, d0, d1, co, mw2, mb2, ux, ua, uc, ub1, uw2, ub2, g, b,
        rw1, rb1, rw2, rb2):
    return pl.pallas_call(
        _kr_body,
        grid=(_N // _BLK,),
        in_specs=[_rows(_H), _rows(_H), _rows(_H), _rows(1), _rows(1),
                  _rows(1), _full((_H, _H)), _full((1, _H)), _full((_H, _H)),
                  _full((_H, _H)), _full((1, _H)), _full((1, _H)),
                  _full((_H, _H)), _full((1, _H)), _full((1, _H)),
                  _full((1, _H)), _full((_H, _H)), _full((1, _H)),
                  _full((_H, 1)), _full((1, 1))],
        out_specs=[_rows(1)],
        out_shape=[jax.ShapeDtypeStruct((_N, 1), _f32)],
    )(x, s0, s1, d0, d1, co, mw2, mb2, ux, ua, uc, ub1, uw2, ub2, g, b,
      rw1, rb1, rw2, rb2)[0]


def _split_layer(L):
    mw1 = L["mw1"]
    return dict(
        ws=mw1[:_H], wd=mw1[_H:2 * _H], we=mw1[2 * _H:2 * _H + _ED],
        wcs=mw1[2 * _H + _ED:2 * _H + _ED + 1],
        wcd=mw1[2 * _H + _ED + 1:2 * _H + _ED + 2],
        mb1=L["mb1"].reshape(1, _H), mw2=L["mw2"],
        mb2=L["mb2"].reshape(1, _H),
        ux=L["uw1"][:_H], ua=L["uw1"][_H:2 * _H],
        uc=L["uw1"][2 * _H:2 * _H + 1],
        ub1=L["ub1"].reshape(1, _H), uw2=L["uw2"],
        ub2=L["ub2"].reshape(1, _H),
        g=L["ln_g"].reshape(1, _H), b=L["ln_b"].reshape(1, _H),
    )


def kernel(node_features, species, edge_index, edge_features, coordination,
           params):
    p = params
    src = edge_index[0].astype(jnp.int32)
    dst = edge_index[1].astype(jnp.int32)
    sp2 = species.reshape(_N, 1).astype(jnp.int32)
    co2 = coordination.reshape(_N, 1).astype(_f32)
    emb_pad = jnp.zeros((_H, _H), _f32).at[:p["emb"].shape[0]].set(p["emb"])
    in_w = p["in_w"]
    wa, wb, wc = in_w[:_H], in_w[_H:2 * _H], in_w[2 * _H:2 * _H + 1]
    inb = p["in_b"].reshape(1, _H)
    L0 = _split_layer(p["layers"][0])
    L1 = _split_layer(p["layers"][1])

    z128 = jnp.zeros((_RPT, _H), _f32)
    ones_h = jnp.ones((_CHUNK, _H), _f32)

    x, a, b = _k0(node_features, sp2, co2, emb_pad, wa, wb, wc, inb,
                  L0["ws"], L0["wd"], L0["wcs"], L0["wcd"], L0["mb1"])
    c0, c1 = _kc(edge_features, L0["we"], L1["we"])

    dd = _sc_deg(dst, z128, ones_h)
    d0, d1 = dd[0, :_N, :1], dd[1, :_N, :1]

    s = _sc_edge(a, b, c0, src, dst, z128)
    s = s[:, :_N]
    x, a, b = _kp(x, s[0], s[1], d0, d1, co2,
                  L0["mw2"], L0["mb2"], L0["ux"], L0["ua"], L0["uc"],
                  L0["ub1"], L0["uw2"], L0["ub2"], L0["g"], L0["b"],
                  L1["ws"], L1["wd"], L1["wcs"], L1["wcd"], L1["mb1"])

    s = _sc_edge(a, b, c1, src, dst, z128)
    s = s[:, :_N]
    out = _kr(x, s[0], s[1], d0, d1, co2,
              L1["mw2"], L1["mb2"], L1["ux"], L1["ua"], L1["uc"],
              L1["ub1"], L1["uw2"], L1["ub2"], L1["g"], L1["b"],
              p["rw1"], p["rb1"].reshape(1, _H), p["rw2"],
              p["rb2"].reshape(1, 1))
    return out.reshape(_N)


# speedup vs baseline: 6.0293x; 1.4607x over previous
import functools

import jax
import jax.numpy as jnp
from jax import lax
from jax.experimental import pallas as pl
from jax.experimental.pallas import tpu as pltpu
from jax.experimental.pallas import tpu_sc as plsc

_N = 10000
_H = 128
_E = 320000
_ED = 16
_NC = 2
_NS = 16
_NW = _NC * _NS
_EPW = _E // _NW
_CHUNK = 40
_NCHUNK = _EPW // _CHUNK
_NP = 10240
_RPT = _NP // _NS
_BLK = 1000
_EBLK = 4000

_f32 = jnp.float32


def _dot(a, b):
    return jnp.dot(a, b, preferred_element_type=_f32,
                   precision=lax.Precision.HIGHEST)


def _full(shape):
    nd = len(shape)
    return pl.BlockSpec(shape, lambda i: (0,) * nd)


def _rows(width):
    return pl.BlockSpec((_BLK, width), lambda i: (i, 0))


def _k0_body(nf, sp, co, emb_pad, wa, wb, wc, inb, ws, wd, wcs, wcd, mb1,
             x_o, a_o, b_o):
    oh = (sp[...] == lax.broadcasted_iota(jnp.int32, (1, _H), 1)).astype(_f32)
    e = _dot(oh, emb_pad[...])
    x = (_dot(nf[...], wa[...])
         + _dot(e, wb[...])
         + co[...] * wc[...] + inb[...])
    x_o[...] = x
    a_o[...] = _dot(x, ws[...]) + co[...] * wcs[...]
    b_o[...] = (_dot(x, wd[...])
                + co[...] * wcd[...] + mb1[...])


def _k0(nf, sp, co, emb_pad, wa, wb, wc, inb, ws, wd, wcs, wcd, mb1):
    return pl.pallas_call(
        _k0_body,
        grid=(_N // _BLK,),
        in_specs=[_rows(_H), _rows(1), _rows(1), _full((_H, _H)),
                  _full((_H, _H)), _full((_H, _H)), _full((1, _H)),
                  _full((1, _H)), _full((_H, _H)), _full((_H, _H)),
                  _full((1, _H)), _full((1, _H)), _full((1, _H))],
        out_specs=[_rows(_H), _rows(_H), _rows(_H)],
        out_shape=[jax.ShapeDtypeStruct((_N, _H), _f32)] * 3,
    )(nf, sp, co, emb_pad, wa, wb, wc, inb, ws, wd, wcs, wcd, mb1)


def _kc_body(ef, we0, we1, c0_o, c1_o):
    c0_o[...] = _dot(ef[...], we0[...])
    c1_o[...] = _dot(ef[...], we1[...])


def _kc(ef, we0, we1):
    return pl.pallas_call(
        _kc_body,
        grid=(_E // _EBLK,),
        in_specs=[pl.BlockSpec((_EBLK, _ED), lambda i: (i, 0)),
                  _full((_ED, _H)), _full((_ED, _H))],
        out_specs=[pl.BlockSpec((_EBLK, _H), lambda i: (i, 0))] * 2,
        out_shape=[jax.ShapeDtypeStruct((_E, _H), _f32)] * 2,
    )(ef, we0, we1)


def _sc_edge_body(a_hbm, b_hbm, c_hbm, src_hbm, dst_hbm, z128,
                  s_out,
                  srcv, dstv, rows_a, rows_b, rows_c,
                  sem_a, sem_b, sem_c, sem_is, sem_id, s_sh):
    c = lax.axis_index("c")
    s = lax.axis_index("s")
    wid = s * _NC + c
    pltpu.sync_copy(z128, s_sh.at[pl.ds(s * _RPT, _RPT)])
    plsc.subcore_barrier()

    base0 = wid * _EPW

    def fire_idx(i, b):
        base = base0 + i * _CHUNK
        pltpu.async_copy(src_hbm.at[pl.ds(base, _CHUNK)], srcv[b], sem_is[b])
        pltpu.async_copy(dst_hbm.at[pl.ds(base, _CHUNK)], dstv[b], sem_id[b])

    def wait_idx(b):
        pltpu.make_async_copy(src_hbm.at[pl.ds(0, _CHUNK)], srcv[b],
                              sem_is[b]).wait()
        pltpu.make_async_copy(dst_hbm.at[pl.ds(0, _CHUNK)], dstv[b],
                              sem_id[b]).wait()

    def fire_rows(i, b):
        base = base0 + i * _CHUNK
        pltpu.async_copy(a_hbm.at[srcv[b]], rows_a[b], sem_a[b])
        pltpu.async_copy(b_hbm.at[dstv[b]], rows_b[b], sem_b[b])
        pltpu.async_copy(c_hbm.at[pl.ds(base, _CHUNK)], rows_c[b], sem_c[b])

    def wait_rows(b):
        pltpu.make_async_copy(a_hbm.at[srcv[b]], rows_a[b], sem_a[b]).wait()
        pltpu.make_async_copy(b_hbm.at[dstv[b]], rows_b[b], sem_b[b]).wait()
        pltpu.make_async_copy(c_hbm.at[pl.ds(0, _CHUNK)], rows_c[b],
                              sem_c[b]).wait()

    fire_idx(0, 0)
    fire_idx(1, 1)
    wait_idx(0)
    fire_rows(0, 0)

    def pair_body(j, carry):
        for b in range(2):
            i = 2 * j + b
            nb = 1 - b

            @pl.when(i + 1 < _NCHUNK)
            def _():
                wait_idx(nb)
                fire_rows(i + 1, nb)

            wait_rows(b)

            def row_body(r, carry2):
                for k in range(_H // 16):
                    sl = pl.ds(k * 16, 16)
                    t = rows_a[b][r, sl] + rows_b[b][r, sl] + rows_c[b][r, sl]
                    rows_c[b][r, sl] = t / (1.0 + jnp.exp(-t))
                return carry2

            lax.fori_loop(0, _CHUNK, row_body, 0)
            pltpu.sync_copy(rows_c[b], s_sh.at[dstv[b]], add=True)

            @pl.when(i + 2 < _NCHUNK)
            def _():
                fire_idx(i + 2, b)
        return carry

    lax.fori_loop(0, _NCHUNK // 2, pair_body, 0)
    plsc.subcore_barrier()
    pltpu.sync_copy(s_sh.at[pl.ds(s * _RPT, _RPT)],
                    s_out.at[c, pl.ds(s * _RPT, _RPT)])


_sc_edge = functools.partial(
    pl.kernel,
    mesh=plsc.VectorSubcoreMesh(core_axis_name="c", subcore_axis_name="s",
                                num_cores=_NC, num_subcores=_NS),
    out_type=jax.ShapeDtypeStruct((_NC, _NP, _H), _f32),
    scratch_types=[
        [pltpu.VMEM((_CHUNK,), jnp.int32)] * 2,
        [pltpu.VMEM((_CHUNK,), jnp.int32)] * 2,
        [pltpu.VMEM((_CHUNK, _H), _f32)] * 2,
        [pltpu.VMEM((_CHUNK, _H), _f32)] * 2,
        [pltpu.VMEM((_CHUNK, _H), _f32)] * 2,
        [pltpu.SemaphoreType.DMA] * 2,
        [pltpu.SemaphoreType.DMA] * 2,
        [pltpu.SemaphoreType.DMA] * 2,
        [pltpu.SemaphoreType.DMA] * 2,
        [pltpu.SemaphoreType.DMA] * 2,
        pltpu.VMEM_SHARED((_NP, _H), _f32),
    ],
)(_sc_edge_body)


def _sc_deg_body(dst_hbm, z128, ones_h, d_out, dstv, onesv, d_sh):
    c = lax.axis_index("c")
    s = lax.axis_index("s")
    wid = s * _NC + c
    pltpu.sync_copy(z128, d_sh.at[pl.ds(s * _RPT, _RPT)])
    pltpu.sync_copy(ones_h, onesv)
    plsc.subcore_barrier()

    base0 = wid * _EPW

    def chunk_body(i, carry):
        base = base0 + i * _CHUNK
        pltpu.sync_copy(dst_hbm.at[pl.ds(base, _CHUNK)], dstv)
        pltpu.sync_copy(onesv, d_sh.at[dstv], add=True)
        return carry

    lax.fori_loop(0, _NCHUNK, chunk_body, 0)
    plsc.subcore_barrier()
    pltpu.sync_copy(d_sh.at[pl.ds(s * _RPT, _RPT)],
                    d_out.at[c, pl.ds(s * _RPT, _RPT)])


_sc_deg = functools.partial(
    pl.kernel,
    mesh=plsc.VectorSubcoreMesh(core_axis_name="c", subcore_axis_name="s",
                                num_cores=_NC, num_subcores=_NS),
    out_type=jax.ShapeDtypeStruct((_NC, _NP, _H), _f32),
    scratch_types=[
        pltpu.VMEM((_CHUNK,), jnp.int32),
        pltpu.VMEM((_CHUNK, _H), _f32),
        pltpu.VMEM_SHARED((_NP, _H), _f32),
    ],
)(_sc_deg_body)


def _post_common(x, s0, s1, d0, d1, co, mw2, mb2, ux, ua, uc, ub1, uw2, ub2,
                 g, b):
    agg = (_dot(s0[...] + s1[...], mw2[...])
           + (d0[...] + d1[...]) * mb2[...])
    t = (_dot(x[...], ux[...])
         + _dot(agg, ua[...])
         + co[...] * uc[...] + ub1[...])
    h = t * jax.nn.sigmoid(t)
    y = x[...] + _dot(h, uw2[...]) + ub2[...]
    mu = jnp.mean(y, axis=-1, keepdims=True)
    yc = y - mu
    var = jnp.mean(yc * yc, axis=-1, keepdims=True)
    return yc * lax.rsqrt(var + 1e-5) * g[...] + b[...]


def _kp_body(x, s0, s1, d0, d1, co, mw2, mb2, ux, ua, uc, ub1, uw2, ub2, g, b,
             ws, wd, wcs, wcd, mb1, x_o, a_o, b_o):
    xn = _post_common(x, s0, s1, d0, d1, co, mw2, mb2, ux, ua, uc, ub1, uw2,
                      ub2, g, b)
    x_o[...] = xn
    a_o[...] = _dot(xn, ws[...]) + co[...] * wcs[...]
    b_o[...] = (_dot(xn, wd[...])
                + co[...] * wcd[...] + mb1[...])


def _kp(x, s0, s1, d0, d1, co, mw2, mb2, ux, ua, uc, ub1, uw2, ub2, g, b,
        ws, wd, wcs, wcd, mb1):
    return pl.pallas_call(
        _kp_body,
        grid=(_N // _BLK,),
        in_specs=[_rows(_H), _rows(_H), _rows(_H), _rows(1), _rows(1),
                  _rows(1), _full((_H, _H)), _full((1, _H)), _full((_H, _H)),
                  _full((_H, _H)), _full((1, _H)), _full((1, _H)),
                  _full((_H, _H)), _full((1, _H)), _full((1, _H)),
                  _full((1, _H)), _full((_H, _H)), _full((_H, _H)),
                  _full((1, _H)), _full((1, _H)), _full((1, _H))],
        out_specs=[_rows(_H), _rows(_H), _rows(_H)],
        out_shape=[jax.ShapeDtypeStruct((_N, _H), _f32)] * 3,
    )(x, s0, s1, d0, d1, co, mw2, mb2, ux, ua, uc, ub1, uw2, ub2, g, b,
      ws, wd, wcs, wcd, mb1)


def _kr_body(x, s0, s1, d0, d1, co, mw2, mb2, ux, ua, uc, ub1, uw2, ub2, g, b,
             rw1, rb1, rw2, rb2, o_o):
    xn = _post_common(x, s0, s1, d0, d1, co, mw2, mb2, ux, ua, uc, ub1, uw2,
                      ub2, g, b)
    t2 = _dot(xn, rw1[...]) + rb1[...]
    h2 = t2 * jax.nn.sigmoid(t2)
    o_o[...] = _dot(h2, rw2[...]) + rb2[...]


def _kr(x, s0, s1, d0, d1, co, mw2, mb2, ux, ua, uc, ub1, uw2, ub2, g, b,
        rw1, rb1, rw2, rb2):
    return pl.pallas_call(
        _kr_body,
        grid=(_N // _BLK,),
        in_specs=[_rows(_H), _rows(_H), _rows(_H), _rows(1), _rows(1),
                  _rows(1), _full((_H, _H)), _full((1, _H)), _full((_H, _H)),
                  _full((_H, _H)), _full((1, _H)), _full((1, _H)),
                  _full((_H, _H)), _full((1, _H)), _full((1, _H)),
                  _full((1, _H)), _full((_H, _H)), _full((1, _H)),
                  _full((_H, 1)), _full((1, 1))],
        out_specs=[_rows(1)],
        out_shape=[jax.ShapeDtypeStruct((_N, 1), _f32)],
    )(x, s0, s1, d0, d1, co, mw2, mb2, ux, ua, uc, ub1, uw2, ub2, g, b,
      rw1, rb1, rw2, rb2)[0]


def _split_layer(L):
    mw1 = L["mw1"]
    return dict(
        ws=mw1[:_H], wd=mw1[_H:2 * _H], we=mw1[2 * _H:2 * _H + _ED],
        wcs=mw1[2 * _H + _ED:2 * _H + _ED + 1],
        wcd=mw1[2 * _H + _ED + 1:2 * _H + _ED + 2],
        mb1=L["mb1"].reshape(1, _H), mw2=L["mw2"],
        mb2=L["mb2"].reshape(1, _H),
        ux=L["uw1"][:_H], ua=L["uw1"][_H:2 * _H],
        uc=L["uw1"][2 * _H:2 * _H + 1],
        ub1=L["ub1"].reshape(1, _H), uw2=L["uw2"],
        ub2=L["ub2"].reshape(1, _H),
        g=L["ln_g"].reshape(1, _H), b=L["ln_b"].reshape(1, _H),
    )


def kernel(node_features, species, edge_index, edge_features, coordination,
           params):
    p = params
    src = edge_index[0].astype(jnp.int32)
    dst = edge_index[1].astype(jnp.int32)
    sp2 = species.reshape(_N, 1).astype(jnp.int32)
    co2 = coordination.reshape(_N, 1).astype(_f32)
    emb_pad = jnp.zeros((_H, _H), _f32).at[:p["emb"].shape[0]].set(p["emb"])
    in_w = p["in_w"]
    wa, wb, wc = in_w[:_H], in_w[_H:2 * _H], in_w[2 * _H:2 * _H + 1]
    inb = p["in_b"].reshape(1, _H)
    L0 = _split_layer(p["layers"][0])
    L1 = _split_layer(p["layers"][1])

    z128 = jnp.zeros((_RPT, _H), _f32)
    ones_h = jnp.ones((_CHUNK, _H), _f32)

    x, a, b = _k0(node_features, sp2, co2, emb_pad, wa, wb, wc, inb,
                  L0["ws"], L0["wd"], L0["wcs"], L0["wcd"], L0["mb1"])
    c0, c1 = _kc(edge_features, L0["we"], L1["we"])

    dd = _sc_deg(dst, z128, ones_h)
    d0, d1 = dd[0, :_N, :1], dd[1, :_N, :1]

    s = _sc_edge(a, b, c0, src, dst, z128)
    s = s[:, :_N]
    x, a, b = _kp(x, s[0], s[1], d0, d1, co2,
                  L0["mw2"], L0["mb2"], L0["ux"], L0["ua"], L0["uc"],
                  L0["ub1"], L0["uw2"], L0["ub2"], L0["g"], L0["b"],
                  L1["ws"], L1["wd"], L1["wcs"], L1["wcd"], L1["mb1"])

    s = _sc_edge(a, b, c1, src, dst, z128)
    s = s[:, :_N]
    out = _kr(x, s[0], s[1], d0, d1, co2,
              L1["mw2"], L1["mb2"], L1["ux"], L1["ua"], L1["uc"],
              L1["ub1"], L1["uw2"], L1["ub2"], L1["g"], L1["b"],
              p["rw1"], p["rb1"].reshape(1, _H), p["rw2"],
              p["rb2"].reshape(1, 1))
    return out.reshape(_N)


# speedup vs baseline: 6.9880x; 1.1590x over previous
import functools

import jax
import jax.numpy as jnp
from jax import lax
from jax.experimental import pallas as pl
from jax.experimental.pallas import tpu as pltpu
from jax.experimental.pallas import tpu_sc as plsc

_N = 10000
_H = 128
_E = 320000
_ED = 16
_NC = 2
_NS = 16
_NW = _NC * _NS
_EPW = _E // _NW
_CHUNK = 40
_NCHUNK = _EPW // _CHUNK
_NP = 10240
_RPT = _NP // _NS
_BLK = 1000
_EBLK = 4000

_f32 = jnp.float32


def _dot(a, b):
    return jnp.dot(a, b, preferred_element_type=_f32,
                   precision=lax.Precision.HIGHEST)


def _full(shape):
    nd = len(shape)
    return pl.BlockSpec(shape, lambda i: (0,) * nd)


def _rows(width):
    return pl.BlockSpec((_BLK, width), lambda i: (i, 0))


def _k0_body(nf, sp, co, emb_pad, wa, wb, wc, inb, ws, wd, wcs, wcd, mb1,
             x_o, a_o, b_o):
    oh = (sp[...] == lax.broadcasted_iota(jnp.int32, (1, _H), 1)).astype(_f32)
    e = _dot(oh, emb_pad[...])
    x = (_dot(nf[...], wa[...])
         + _dot(e, wb[...])
         + co[...] * wc[...] + inb[...])
    x_o[...] = x
    a_o[...] = _dot(x, ws[...]) + co[...] * wcs[...]
    b_o[...] = (_dot(x, wd[...])
                + co[...] * wcd[...] + mb1[...])


def _k0(nf, sp, co, emb_pad, wa, wb, wc, inb, ws, wd, wcs, wcd, mb1):
    return pl.pallas_call(
        _k0_body,
        grid=(_N // _BLK,),
        in_specs=[_rows(_H), _rows(1), _rows(1), _full((_H, _H)),
                  _full((_H, _H)), _full((_H, _H)), _full((1, _H)),
                  _full((1, _H)), _full((_H, _H)), _full((_H, _H)),
                  _full((1, _H)), _full((1, _H)), _full((1, _H))],
        out_specs=[_rows(_H), _rows(_H), _rows(_H)],
        out_shape=[jax.ShapeDtypeStruct((_N, _H), _f32)] * 3,
    )(nf, sp, co, emb_pad, wa, wb, wc, inb, ws, wd, wcs, wcd, mb1)


def _kc_body(ef, we0, we1, c0_o, c1_o):
    c0_o[...] = jnp.dot(ef[...], we0[...], preferred_element_type=_f32)
    c1_o[...] = jnp.dot(ef[...], we1[...], preferred_element_type=_f32)


def _kc(ef, we0, we1):
    return pl.pallas_call(
        _kc_body,
        grid=(_E // _EBLK,),
        in_specs=[pl.BlockSpec((_EBLK, _ED), lambda i: (i, 0)),
                  _full((_ED, _H)), _full((_ED, _H))],
        out_specs=[pl.BlockSpec((_EBLK, _H), lambda i: (i, 0))] * 2,
        out_shape=[jax.ShapeDtypeStruct((_E, _H), _f32)] * 2,
    )(ef, we0, we1)


def _sc_edge_body(a_hbm, b_hbm, c_hbm, src_hbm, dst_hbm, z128,
                  s_out,
                  srcv, dstv, rows_a, rows_b, rows_c,
                  sem_a, sem_b, sem_c, sem_is, sem_id, s_sh):
    c = lax.axis_index("c")
    s = lax.axis_index("s")
    wid = s * _NC + c
    pltpu.sync_copy(z128, s_sh.at[pl.ds(s * _RPT, _RPT)])
    plsc.subcore_barrier()

    base0 = wid * _EPW

    def fire_idx(i, b):
        base = base0 + i * _CHUNK
        pltpu.async_copy(src_hbm.at[pl.ds(base, _CHUNK)], srcv[b], sem_is[b])
        pltpu.async_copy(dst_hbm.at[pl.ds(base, _CHUNK)], dstv[b], sem_id[b])

    def wait_idx(b):
        pltpu.make_async_copy(src_hbm.at[pl.ds(0, _CHUNK)], srcv[b],
                              sem_is[b]).wait()
        pltpu.make_async_copy(dst_hbm.at[pl.ds(0, _CHUNK)], dstv[b],
                              sem_id[b]).wait()

    def fire_rows(i, b):
        base = base0 + i * _CHUNK
        pltpu.async_copy(a_hbm.at[srcv[b]], rows_a[b], sem_a[b])
        pltpu.async_copy(b_hbm.at[dstv[b]], rows_b[b], sem_b[b])
        pltpu.async_copy(c_hbm.at[pl.ds(base, _CHUNK)], rows_c[b], sem_c[b])

    def wait_rows(b):
        pltpu.make_async_copy(a_hbm.at[srcv[b]], rows_a[b], sem_a[b]).wait()
        pltpu.make_async_copy(b_hbm.at[dstv[b]], rows_b[b], sem_b[b]).wait()
        pltpu.make_async_copy(c_hbm.at[pl.ds(0, _CHUNK)], rows_c[b],
                              sem_c[b]).wait()

    fire_idx(0, 0)
    fire_idx(1, 1)
    wait_idx(0)
    fire_rows(0, 0)

    def pair_body(j, carry):
        for b in range(2):
            i = 2 * j + b
            nb = 1 - b

            @pl.when(i + 1 < _NCHUNK)
            def _():
                wait_idx(nb)
                fire_rows(i + 1, nb)

            wait_rows(b)

            @plsc.parallel_loop(0, _CHUNK, 1, unroll=2)
            def row_body(r):
                for k in range(_H // 16):
                    sl = pl.ds(k * 16, 16)
                    t = rows_a[b][r, sl] + rows_b[b][r, sl] + rows_c[b][r, sl]
                    rows_c[b][r, sl] = t / (1.0 + jnp.exp(-t))

            pltpu.sync_copy(rows_c[b], s_sh.at[dstv[b]], add=True)

            @pl.when(i + 2 < _NCHUNK)
            def _():
                fire_idx(i + 2, b)
        return carry

    lax.fori_loop(0, _NCHUNK // 2, pair_body, 0)
    plsc.subcore_barrier()
    pltpu.sync_copy(s_sh.at[pl.ds(s * _RPT, _RPT)],
                    s_out.at[c, pl.ds(s * _RPT, _RPT)])


_sc_edge = functools.partial(
    pl.kernel,
    mesh=plsc.VectorSubcoreMesh(core_axis_name="c", subcore_axis_name="s",
                                num_cores=_NC, num_subcores=_NS),
    out_type=jax.ShapeDtypeStruct((_NC, _NP, _H), _f32),
    scratch_types=[
        [pltpu.VMEM((_CHUNK,), jnp.int32)] * 2,
        [pltpu.VMEM((_CHUNK,), jnp.int32)] * 2,
        [pltpu.VMEM((_CHUNK, _H), _f32)] * 2,
        [pltpu.VMEM((_CHUNK, _H), _f32)] * 2,
        [pltpu.VMEM((_CHUNK, _H), _f32)] * 2,
        [pltpu.SemaphoreType.DMA] * 2,
        [pltpu.SemaphoreType.DMA] * 2,
        [pltpu.SemaphoreType.DMA] * 2,
        [pltpu.SemaphoreType.DMA] * 2,
        [pltpu.SemaphoreType.DMA] * 2,
        pltpu.VMEM_SHARED((_NP, _H), _f32),
    ],
)(_sc_edge_body)


def _sc_deg_body(dst_hbm, z128, ones_h, d_out, dstv, onesv, sem_id, d_sh):
    c = lax.axis_index("c")
    s = lax.axis_index("s")
    wid = s * _NC + c
    pltpu.sync_copy(z128, d_sh.at[pl.ds(s * _RPT, _RPT)])
    pltpu.sync_copy(ones_h, onesv)
    plsc.subcore_barrier()

    base0 = wid * _EPW

    def fire_idx(i, b):
        base = base0 + i * _CHUNK
        pltpu.async_copy(dst_hbm.at[pl.ds(base, _CHUNK)], dstv[b], sem_id[b])

    def wait_idx(b):
        pltpu.make_async_copy(dst_hbm.at[pl.ds(0, _CHUNK)], dstv[b],
                              sem_id[b]).wait()

    fire_idx(0, 0)
    fire_idx(1, 1)

    def pair_body(j, carry):
        for b in range(2):
            i = 2 * j + b
            wait_idx(b)
            pltpu.sync_copy(onesv, d_sh.at[dstv[b]], add=True)

            @pl.when(i + 2 < _NCHUNK)
            def _():
                fire_idx(i + 2, b)
        return carry

    lax.fori_loop(0, _NCHUNK // 2, pair_body, 0)
    plsc.subcore_barrier()
    pltpu.sync_copy(d_sh.at[pl.ds(s * _RPT, _RPT)],
                    d_out.at[c, pl.ds(s * _RPT, _RPT)])


_sc_deg = functools.partial(
    pl.kernel,
    mesh=plsc.VectorSubcoreMesh(core_axis_name="c", subcore_axis_name="s",
                                num_cores=_NC, num_subcores=_NS),
    out_type=jax.ShapeDtypeStruct((_NC, _NP, _H), _f32),
    scratch_types=[
        [pltpu.VMEM((_CHUNK,), jnp.int32)] * 2,
        pltpu.VMEM((_CHUNK, _H), _f32),
        [pltpu.SemaphoreType.DMA] * 2,
        pltpu.VMEM_SHARED((_NP, _H), _f32),
    ],
)(_sc_deg_body)


def _post_common(x, s0, s1, d0, d1, co, mw2, mb2, ux, ua, uc, ub1, uw2, ub2,
                 g, b):
    agg = (_dot(s0[...] + s1[...], mw2[...])
           + (d0[...] + d1[...]) * mb2[...])
    t = (_dot(x[...], ux[...])
         + _dot(agg, ua[...])
         + co[...] * uc[...] + ub1[...])
    h = t * jax.nn.sigmoid(t)
    y = x[...] + _dot(h, uw2[...]) + ub2[...]
    mu = jnp.mean(y, axis=-1, keepdims=True)
    yc = y - mu
    var = jnp.mean(yc * yc, axis=-1, keepdims=True)
    return yc * lax.rsqrt(var + 1e-5) * g[...] + b[...]


def _kp_body(x, s0, s1, d0, d1, co, mw2, mb2, ux, ua, uc, ub1, uw2, ub2, g, b,
             ws, wd, wcs, wcd, mb1, x_o, a_o, b_o):
    xn = _post_common(x, s0, s1, d0, d1, co, mw2, mb2, ux, ua, uc, ub1, uw2,
                      ub2, g, b)
    x_o[...] = xn
    a_o[...] = _dot(xn, ws[...]) + co[...] * wcs[...]
    b_o[...] = (_dot(xn, wd[...])
                + co[...] * wcd[...] + mb1[...])


def _kp(x, s0, s1, d0, d1, co, mw2, mb2, ux, ua, uc, ub1, uw2, ub2, g, b,
        ws, wd, wcs, wcd, mb1):
    return pl.pallas_call(
        _kp_body,
        grid=(_N // _BLK,),
        in_specs=[_rows(_H), _rows(_H), _rows(_H), _rows(1), _rows(1),
                  _rows(1), _full((_H, _H)), _full((1, _H)), _full((_H, _H)),
                  _full((_H, _H)), _full((1, _H)), _full((1, _H)),
                  _full((_H, _H)), _full((1, _H)), _full((1, _H)),
                  _full((1, _H)), _full((_H, _H)), _full((_H, _H)),
                  _full((1, _H)), _full((1, _H)), _full((1, _H))],
        out_specs=[_rows(_H), _rows(_H), _rows(_H)],
        out_shape=[jax.ShapeDtypeStruct((_N, _H), _f32)] * 3,
    )(x, s0, s1, d0, d1, co, mw2, mb2, ux, ua, uc, ub1, uw2, ub2, g, b,
      ws, wd, wcs, wcd, mb1)


def _kr_body(x, s0, s1, d0, d1, co, mw2, mb2, ux, ua, uc, ub1, uw2, ub2, g, b,
             rw1, rb1, rw2, rb2, o_o):
    xn = _post_common(x, s0, s1, d0, d1, co, mw2, mb2, ux, ua, uc, ub1, uw2,
                      ub2, g, b)
    t2 = _dot(xn, rw1[...]) + rb1[...]
    h2 = t2 * jax.nn.sigmoid(t2)
    o_o[...] = _dot(h2, rw2[...]) + rb2[...]


def _kr(x, s0, s1, d0, d1, co, mw2, mb2, ux, ua, uc, ub1, uw2, ub2, g, b,
        rw1, rb1, rw2, rb2):
    return pl.pallas_call(
        _kr_body,
        grid=(_N // _BLK,),
        in_specs=[_rows(_H), _rows(_H), _rows(_H), _rows(1), _rows(1),
                  _rows(1), _full((_H, _H)), _full((1, _H)), _full((_H, _H)),
                  _full((_H, _H)), _full((1, _H)), _full((1, _H)),
                  _full((_H, _H)), _full((1, _H)), _full((1, _H)),
                  _full((1, _H)), _full((_H, _H)), _full((1, _H)),
                  _full((_H, 1)), _full((1, 1))],
        out_specs=[_rows(1)],
        out_shape=[jax.ShapeDtypeStruct((_N, 1), _f32)],
    )(x, s0, s1, d0, d1, co, mw2, mb2, ux, ua, uc, ub1, uw2, ub2, g, b,
      rw1, rb1, rw2, rb2)[0]


def _split_layer(L):
    mw1 = L["mw1"]
    return dict(
        ws=mw1[:_H], wd=mw1[_H:2 * _H], we=mw1[2 * _H:2 * _H + _ED],
        wcs=mw1[2 * _H + _ED:2 * _H + _ED + 1],
        wcd=mw1[2 * _H + _ED + 1:2 * _H + _ED + 2],
        mb1=L["mb1"].reshape(1, _H), mw2=L["mw2"],
        mb2=L["mb2"].reshape(1, _H),
        ux=L["uw1"][:_H], ua=L["uw1"][_H:2 * _H],
        uc=L["uw1"][2 * _H:2 * _H + 1],
        ub1=L["ub1"].reshape(1, _H), uw2=L["uw2"],
        ub2=L["ub2"].reshape(1, _H),
        g=L["ln_g"].reshape(1, _H), b=L["ln_b"].reshape(1, _H),
    )


def kernel(node_features, species, edge_index, edge_features, coordination,
           params):
    p = params
    src = edge_index[0].astype(jnp.int32)
    dst = edge_index[1].astype(jnp.int32)
    sp2 = species.reshape(_N, 1).astype(jnp.int32)
    co2 = coordination.reshape(_N, 1).astype(_f32)
    emb_pad = jnp.zeros((_H, _H), _f32).at[:p["emb"].shape[0]].set(p["emb"])
    in_w = p["in_w"]
    wa, wb, wc = in_w[:_H], in_w[_H:2 * _H], in_w[2 * _H:2 * _H + 1]
    inb = p["in_b"].reshape(1, _H)
    L0 = _split_layer(p["layers"][0])
    L1 = _split_layer(p["layers"][1])

    z128 = jnp.zeros((_RPT, _H), _f32)
    ones_h = jnp.ones((_CHUNK, _H), _f32)

    x, a, b = _k0(node_features, sp2, co2, emb_pad, wa, wb, wc, inb,
                  L0["ws"], L0["wd"], L0["wcs"], L0["wcd"], L0["mb1"])
    c0, c1 = _kc(edge_features, L0["we"], L1["we"])

    dd = _sc_deg(dst, z128, ones_h)
    d0, d1 = dd[0, :_N, :1], dd[1, :_N, :1]

    s = _sc_edge(a, b, c0, src, dst, z128)
    s = s[:, :_N]
    x, a, b = _kp(x, s[0], s[1], d0, d1, co2,
                  L0["mw2"], L0["mb2"], L0["ux"], L0["ua"], L0["uc"],
                  L0["ub1"], L0["uw2"], L0["ub2"], L0["g"], L0["b"],
                  L1["ws"], L1["wd"], L1["wcs"], L1["wcd"], L1["mb1"])

    s = _sc_edge(a, b, c1, src, dst, z128)
    s = s[:, :_N]
    out = _kr(x, s[0], s[1], d0, d1, co2,
              L1["mw2"], L1["mb2"], L1["ux"], L1["ua"], L1["uc"],
              L1["ub1"], L1["uw2"], L1["ub2"], L1["g"], L1["b"],
              p["rw1"], p["rb1"].reshape(1, _H), p["rw2"],
              p["rb2"].reshape(1, 1))
    return out.reshape(_N)


# speedup vs baseline: 7.8576x; 1.1244x over previous
import functools

import jax
import jax.numpy as jnp
from jax import lax
from jax.experimental import pallas as pl
from jax.experimental.pallas import tpu as pltpu
from jax.experimental.pallas import tpu_sc as plsc

_N = 10000
_H = 128
_E = 320000
_ED = 16
_NC = 2
_NS = 16
_NW = _NC * _NS
_EPW = _E // _NW
_CHUNK = 40
_NCHUNK = _EPW // _CHUNK
_NP = 10240
_RPT = _NP // _NS
_BLK = 1000
_EBLK = 4000

_f32 = jnp.float32


def _dot(a, b):
    return jnp.dot(a, b, preferred_element_type=_f32)


def _full(shape):
    nd = len(shape)
    return pl.BlockSpec(shape, lambda i: (0,) * nd)


def _rows(width):
    return pl.BlockSpec((_BLK, width), lambda i: (i, 0))


def _k0_body(nf, sp, co, emb_pad, wa, wb, wc, inb, ws, wd, wcs, wcd, mb1,
             x_o, a_o, b_o):
    oh = (sp[...] == lax.broadcasted_iota(jnp.int32, (1, _H), 1)).astype(_f32)
    e = _dot(oh, emb_pad[...])
    x = (_dot(nf[...], wa[...])
         + _dot(e, wb[...])
         + co[...] * wc[...] + inb[...])
    x_o[...] = x
    a_o[...] = _dot(x, ws[...]) + co[...] * wcs[...]
    b_o[...] = (_dot(x, wd[...])
                + co[...] * wcd[...] + mb1[...])


def _k0(nf, sp, co, emb_pad, wa, wb, wc, inb, ws, wd, wcs, wcd, mb1):
    return pl.pallas_call(
        _k0_body,
        grid=(_N // _BLK,),
        in_specs=[_rows(_H), _rows(1), _rows(1), _full((_H, _H)),
                  _full((_H, _H)), _full((_H, _H)), _full((1, _H)),
                  _full((1, _H)), _full((_H, _H)), _full((_H, _H)),
                  _full((1, _H)), _full((1, _H)), _full((1, _H))],
        out_specs=[_rows(_H), _rows(_H), _rows(_H)],
        out_shape=[jax.ShapeDtypeStruct((_N, _H), _f32)] * 3,
    )(nf, sp, co, emb_pad, wa, wb, wc, inb, ws, wd, wcs, wcd, mb1)


def _kc_body(ef, we, c_o):
    c_o[...] = jnp.dot(ef[...], we[...], preferred_element_type=_f32)


def _kc(ef, we):
    return pl.pallas_call(
        _kc_body,
        grid=(_E // _EBLK,),
        in_specs=[pl.BlockSpec((_EBLK, _ED), lambda i: (i, 0)),
                  _full((_ED, _H))],
        out_specs=[pl.BlockSpec((_EBLK, _H), lambda i: (i, 0))],
        out_shape=[jax.ShapeDtypeStruct((_E, _H), _f32)],
    )(ef, we)[0]


def _sc_edge_body(a_hbm, b_hbm, c_hbm, src_hbm, dst_hbm, z128,
                  s_out,
                  srcv, dstv, rows_a, rows_b, rows_c,
                  sem_a, sem_b, sem_c, sem_is, sem_id, s_sh):
    c = lax.axis_index("c")
    s = lax.axis_index("s")
    wid = s * _NC + c
    pltpu.sync_copy(z128, s_sh.at[pl.ds(s * _RPT, _RPT)])
    plsc.subcore_barrier()

    base0 = wid * _EPW

    def fire_idx(i, b):
        base = base0 + i * _CHUNK
        pltpu.async_copy(src_hbm.at[pl.ds(base, _CHUNK)], srcv[b], sem_is[b])
        pltpu.async_copy(dst_hbm.at[pl.ds(base, _CHUNK)], dstv[b], sem_id[b])

    def wait_idx(b):
        pltpu.make_async_copy(src_hbm.at[pl.ds(0, _CHUNK)], srcv[b],
                              sem_is[b]).wait()
        pltpu.make_async_copy(dst_hbm.at[pl.ds(0, _CHUNK)], dstv[b],
                              sem_id[b]).wait()

    def fire_rows(i, b):
        base = base0 + i * _CHUNK
        pltpu.async_copy(a_hbm.at[srcv[b]], rows_a[b], sem_a[b])
        pltpu.async_copy(b_hbm.at[dstv[b]], rows_b[b], sem_b[b])
        pltpu.async_copy(c_hbm.at[pl.ds(base, _CHUNK)], rows_c[b], sem_c[b])

    def wait_rows(b):
        pltpu.make_async_copy(a_hbm.at[srcv[b]], rows_a[b], sem_a[b]).wait()
        pltpu.make_async_copy(b_hbm.at[dstv[b]], rows_b[b], sem_b[b]).wait()
        pltpu.make_async_copy(c_hbm.at[pl.ds(0, _CHUNK)], rows_c[b],
                              sem_c[b]).wait()

    fire_idx(0, 0)
    fire_idx(1, 1)
    wait_idx(0)
    fire_rows(0, 0)

    def pair_body(j, carry):
        for b in range(2):
            i = 2 * j + b
            nb = 1 - b

            @pl.when(i + 1 < _NCHUNK)
            def _():
                wait_idx(nb)
                fire_rows(i + 1, nb)

            wait_rows(b)

            @plsc.parallel_loop(0, _CHUNK, 1, unroll=2)
            def row_body(r):
                for k in range(_H // 16):
                    sl = pl.ds(k * 16, 16)
                    t = rows_a[b][r, sl] + rows_b[b][r, sl] + rows_c[b][r, sl]
                    rows_c[b][r, sl] = t / (1.0 + jnp.exp(-t))

            pltpu.sync_copy(rows_c[b], s_sh.at[dstv[b]], add=True)

            @pl.when(i + 2 < _NCHUNK)
            def _():
                fire_idx(i + 2, b)
        return carry

    lax.fori_loop(0, _NCHUNK // 2, pair_body, 0)
    plsc.subcore_barrier()
    pltpu.sync_copy(s_sh.at[pl.ds(s * _RPT, _RPT)],
                    s_out.at[c, pl.ds(s * _RPT, _RPT)])


_sc_edge = functools.partial(
    pl.kernel,
    mesh=plsc.VectorSubcoreMesh(core_axis_name="c", subcore_axis_name="s",
                                num_cores=_NC, num_subcores=_NS),
    out_type=jax.ShapeDtypeStruct((_NC, _NP, _H), _f32),
    scratch_types=[
        [pltpu.VMEM((_CHUNK,), jnp.int32)] * 2,
        [pltpu.VMEM((_CHUNK,), jnp.int32)] * 2,
        [pltpu.VMEM((_CHUNK, _H), _f32)] * 2,
        [pltpu.VMEM((_CHUNK, _H), _f32)] * 2,
        [pltpu.VMEM((_CHUNK, _H), _f32)] * 2,
        [pltpu.SemaphoreType.DMA] * 2,
        [pltpu.SemaphoreType.DMA] * 2,
        [pltpu.SemaphoreType.DMA] * 2,
        [pltpu.SemaphoreType.DMA] * 2,
        [pltpu.SemaphoreType.DMA] * 2,
        pltpu.VMEM_SHARED((_NP, _H), _f32),
    ],
)(_sc_edge_body)


def _sc_deg_body(dst_hbm, z128, ones_h, d_out, dstv, onesv, sem_id, d_sh):
    c = lax.axis_index("c")
    s = lax.axis_index("s")
    wid = s * _NC + c
    pltpu.sync_copy(z128, d_sh.at[pl.ds(s * _RPT, _RPT)])
    pltpu.sync_copy(ones_h, onesv)
    plsc.subcore_barrier()

    base0 = wid * _EPW

    def fire_idx(i, b):
        base = base0 + i * _CHUNK
        pltpu.async_copy(dst_hbm.at[pl.ds(base, _CHUNK)], dstv[b], sem_id[b])

    def wait_idx(b):
        pltpu.make_async_copy(dst_hbm.at[pl.ds(0, _CHUNK)], dstv[b],
                              sem_id[b]).wait()

    fire_idx(0, 0)
    fire_idx(1, 1)

    def pair_body(j, carry):
        for b in range(2):
            i = 2 * j + b
            wait_idx(b)
            pltpu.sync_copy(onesv, d_sh.at[dstv[b]], add=True)

            @pl.when(i + 2 < _NCHUNK)
            def _():
                fire_idx(i + 2, b)
        return carry

    lax.fori_loop(0, _NCHUNK // 2, pair_body, 0)
    plsc.subcore_barrier()
    pltpu.sync_copy(d_sh.at[pl.ds(s * _RPT, _RPT)],
                    d_out.at[c, pl.ds(s * _RPT, _RPT)])


_sc_deg = functools.partial(
    pl.kernel,
    mesh=plsc.VectorSubcoreMesh(core_axis_name="c", subcore_axis_name="s",
                                num_cores=_NC, num_subcores=_NS),
    out_type=jax.ShapeDtypeStruct((_NC, _NP, _H), _f32),
    scratch_types=[
        [pltpu.VMEM((_CHUNK,), jnp.int32)] * 2,
        pltpu.VMEM((_CHUNK, _H), _f32),
        [pltpu.SemaphoreType.DMA] * 2,
        pltpu.VMEM_SHARED((_NP, _H), _f32),
    ],
)(_sc_deg_body)


def _post_common(x, s0, s1, d0, d1, co, mw2, mb2, ux, ua, uc, ub1, uw2, ub2,
                 g, b):
    agg = (_dot(s0[...] + s1[...], mw2[...])
           + (d0[...] + d1[...]) * mb2[...])
    t = (_dot(x[...], ux[...])
         + _dot(agg, ua[...])
         + co[...] * uc[...] + ub1[...])
    h = t * jax.nn.sigmoid(t)
    y = x[...] + _dot(h, uw2[...]) + ub2[...]
    mu = jnp.mean(y, axis=-1, keepdims=True)
    yc = y - mu
    var = jnp.mean(yc * yc, axis=-1, keepdims=True)
    return yc * lax.rsqrt(var + 1e-5) * g[...] + b[...]


def _kp_body(x, s0, s1, d0, d1, co, mw2, mb2, ux, ua, uc, ub1, uw2, ub2, g, b,
             ws, wd, wcs, wcd, mb1, x_o, a_o, b_o):
    xn = _post_common(x, s0, s1, d0, d1, co, mw2, mb2, ux, ua, uc, ub1, uw2,
                      ub2, g, b)
    x_o[...] = xn
    a_o[...] = _dot(xn, ws[...]) + co[...] * wcs[...]
    b_o[...] = (_dot(xn, wd[...])
                + co[...] * wcd[...] + mb1[...])


def _kp(x, s0, s1, d0, d1, co, mw2, mb2, ux, ua, uc, ub1, uw2, ub2, g, b,
        ws, wd, wcs, wcd, mb1):
    return pl.pallas_call(
        _kp_body,
        grid=(_N // _BLK,),
        in_specs=[_rows(_H), _rows(_H), _rows(_H), _rows(1), _rows(1),
                  _rows(1), _full((_H, _H)), _full((1, _H)), _full((_H, _H)),
                  _full((_H, _H)), _full((1, _H)), _full((1, _H)),
                  _full((_H, _H)), _full((1, _H)), _full((1, _H)),
                  _full((1, _H)), _full((_H, _H)), _full((_H, _H)),
                  _full((1, _H)), _full((1, _H)), _full((1, _H))],
        out_specs=[_rows(_H), _rows(_H), _rows(_H)],
        out_shape=[jax.ShapeDtypeStruct((_N, _H), _f32)] * 3,
    )(x, s0, s1, d0, d1, co, mw2, mb2, ux, ua, uc, ub1, uw2, ub2, g, b,
      ws, wd, wcs, wcd, mb1)


def _kr_body(x, s0, s1, d0, d1, co, mw2, mb2, ux, ua, uc, ub1, uw2, ub2, g, b,
             rw1, rb1, rw2, rb2, o_o):
    xn = _post_common(x, s0, s1, d0, d1, co, mw2, mb2, ux, ua, uc, ub1, uw2,
                      ub2, g, b)
    t2 = _dot(xn, rw1[...]) + rb1[...]
    h2 = t2 * jax.nn.sigmoid(t2)
    o_o[...] = _dot(h2, rw2[...]) + rb2[...]


def _kr(x, s0, s1, d0, d1, co, mw2, mb2, ux, ua, uc, ub1, uw2, ub2, g, b,
        rw1, rb1, rw2, rb2):
    return pl.pallas_call(
        _kr_body,
        grid=(_N // _BLK,),
        in_specs=[_rows(_H), _rows(_H), _rows(_H), _rows(1), _rows(1),
                  _rows(1), _full((_H, _H)), _full((1, _H)), _full((_H, _H)),
                  _full((_H, _H)), _full((1, _H)), _full((1, _H)),
                  _full((_H, _H)), _full((1, _H)), _full((1, _H)),
                  _full((1, _H)), _full((_H, _H)), _full((1, _H)),
                  _full((_H, 1)), _full((1, 1))],
        out_specs=[_rows(1)],
        out_shape=[jax.ShapeDtypeStruct((_N, 1), _f32)],
    )(x, s0, s1, d0, d1, co, mw2, mb2, ux, ua, uc, ub1, uw2, ub2, g, b,
      rw1, rb1, rw2, rb2)[0]


def _split_layer(L):
    mw1 = L["mw1"]
    return dict(
        ws=mw1[:_H], wd=mw1[_H:2 * _H], we=mw1[2 * _H:2 * _H + _ED],
        wcs=mw1[2 * _H + _ED:2 * _H + _ED + 1],
        wcd=mw1[2 * _H + _ED + 1:2 * _H + _ED + 2],
        mb1=L["mb1"].reshape(1, _H), mw2=L["mw2"],
        mb2=L["mb2"].reshape(1, _H),
        ux=L["uw1"][:_H], ua=L["uw1"][_H:2 * _H],
        uc=L["uw1"][2 * _H:2 * _H + 1],
        ub1=L["ub1"].reshape(1, _H), uw2=L["uw2"],
        ub2=L["ub2"].reshape(1, _H),
        g=L["ln_g"].reshape(1, _H), b=L["ln_b"].reshape(1, _H),
    )


def kernel(node_features, species, edge_index, edge_features, coordination,
           params):
    p = params
    src = edge_index[0].astype(jnp.int32)
    dst = edge_index[1].astype(jnp.int32)
    sp2 = species.reshape(_N, 1).astype(jnp.int32)
    co2 = coordination.reshape(_N, 1).astype(_f32)
    emb_pad = jnp.zeros((_H, _H), _f32).at[:p["emb"].shape[0]].set(p["emb"])
    in_w = p["in_w"]
    wa, wb, wc = in_w[:_H], in_w[_H:2 * _H], in_w[2 * _H:2 * _H + 1]
    inb = p["in_b"].reshape(1, _H)
    L0 = _split_layer(p["layers"][0])
    L1 = _split_layer(p["layers"][1])

    z128 = jnp.zeros((_RPT, _H), _f32)
    ones_h = jnp.ones((_CHUNK, _H), _f32)

    dd = _sc_deg(dst, z128, ones_h)
    d0, d1 = dd[0, :_N, :1], dd[1, :_N, :1]

    x, a, b = _k0(node_features, sp2, co2, emb_pad, wa, wb, wc, inb,
                  L0["ws"], L0["wd"], L0["wcs"], L0["wcd"], L0["mb1"])
    c0 = _kc(edge_features, L0["we"])

    s = _sc_edge(a, b, c0, src, dst, z128)
    c1 = _kc(edge_features, L1["we"])
    s = s[:, :_N]
    x, a, b = _kp(x, s[0], s[1], d0, d1, co2,
                  L0["mw2"], L0["mb2"], L0["ux"], L0["ua"], L0["uc"],
                  L0["ub1"], L0["uw2"], L0["ub2"], L0["g"], L0["b"],
                  L1["ws"], L1["wd"], L1["wcs"], L1["wcd"], L1["mb1"])

    s = _sc_edge(a, b, c1, src, dst, z128)
    s = s[:, :_N]
    out = _kr(x, s[0], s[1], d0, d1, co2,
              L1["mw2"], L1["mb2"], L1["ux"], L1["ua"], L1["uc"],
              L1["ub1"], L1["uw2"], L1["ub2"], L1["g"], L1["b"],
              p["rw1"], p["rb1"].reshape(1, _H), p["rw2"],
              p["rb2"].reshape(1, 1))
    return out.reshape(_N)


# speedup vs baseline: 8.0258x; 1.0214x over previous
import functools

import jax
import jax.numpy as jnp
from jax import lax
from jax.experimental import pallas as pl
from jax.experimental.pallas import tpu as pltpu
from jax.experimental.pallas import tpu_sc as plsc

_N = 10000
_H = 128
_E = 320000
_ED = 16
_NC = 2
_NS = 16
_NW = _NC * _NS
_EPW = _E // _NW
_CHUNK = 40
_NCHUNK = _EPW // _CHUNK
_NP = 10240
_RPT = _NP // _NS
_BLK = 1000
_EBLK = 4000

_f32 = jnp.float32


def _dot(a, b):
    return jnp.dot(a, b, preferred_element_type=_f32)


def _full(shape):
    nd = len(shape)
    return pl.BlockSpec(shape, lambda i: (0,) * nd)


def _rows(width):
    return pl.BlockSpec((_BLK, width), lambda i: (i, 0))


def _k0_body(nf, sp, co, emb_pad, wa, wb, wc, inb, ws, wd, wcs, wcd, mb1,
             x_o, a_o, b_o):
    oh = (sp[...] == lax.broadcasted_iota(jnp.int32, (1, _H), 1)).astype(_f32)
    e = _dot(oh, emb_pad[...])
    x = (_dot(nf[...], wa[...])
         + _dot(e, wb[...])
         + co[...] * wc[...] + inb[...])
    x_o[...] = x
    a_o[...] = _dot(x, ws[...]) + co[...] * wcs[...]
    b_o[...] = (_dot(x, wd[...])
                + co[...] * wcd[...] + mb1[...])


def _k0(nf, sp, co, emb_pad, wa, wb, wc, inb, ws, wd, wcs, wcd, mb1):
    return pl.pallas_call(
        _k0_body,
        grid=(_N // _BLK,),
        in_specs=[_rows(_H), _rows(1), _rows(1), _full((_H, _H)),
                  _full((_H, _H)), _full((_H, _H)), _full((1, _H)),
                  _full((1, _H)), _full((_H, _H)), _full((_H, _H)),
                  _full((1, _H)), _full((1, _H)), _full((1, _H))],
        out_specs=[_rows(_H), _rows(_H), _rows(_H)],
        out_shape=[jax.ShapeDtypeStruct((_N, _H), _f32)] * 3,
    )(nf, sp, co, emb_pad, wa, wb, wc, inb, ws, wd, wcs, wcd, mb1)


def _kc_body(ef, we, c_o):
    c_o[...] = jnp.dot(ef[...], we[...], preferred_element_type=_f32)


def _kc(ef, we):
    return pl.pallas_call(
        _kc_body,
        grid=(_E // _EBLK,),
        in_specs=[pl.BlockSpec((_EBLK, _ED), lambda i: (i, 0)),
                  _full((_ED, _H))],
        out_specs=[pl.BlockSpec((_EBLK, _H), lambda i: (i, 0))],
        out_shape=[jax.ShapeDtypeStruct((_E, _H), _f32)],
    )(ef, we)[0]


def _sc_edge_body(a_hbm, b_hbm, c_hbm, src_hbm, dst_hbm, z128,
                  s_out,
                  srcv, dstv, dmsg, rows_a, rows_b, rows_c, msg,
                  sem_a, sem_b, sem_c, sem_is, sem_id, sem_s, sem_ds, s_sh):
    c = lax.axis_index("c")
    s = lax.axis_index("s")
    wid = s * _NC + c
    pltpu.sync_copy(z128, s_sh.at[pl.ds(s * _RPT, _RPT)])
    plsc.subcore_barrier()

    base0 = wid * _EPW

    def fire_idx(i, b):
        base = base0 + i * _CHUNK
        pltpu.async_copy(src_hbm.at[pl.ds(base, _CHUNK)], srcv[b], sem_is[b])
        pltpu.async_copy(dst_hbm.at[pl.ds(base, _CHUNK)], dstv[b], sem_id[b])

    def wait_idx(b):
        pltpu.make_async_copy(src_hbm.at[pl.ds(0, _CHUNK)], srcv[b],
                              sem_is[b]).wait()
        pltpu.make_async_copy(dst_hbm.at[pl.ds(0, _CHUNK)], dstv[b],
                              sem_id[b]).wait()

    def fire_rows(i, b):
        base = base0 + i * _CHUNK
        pltpu.async_copy(a_hbm.at[srcv[b]], rows_a[b], sem_a[b])
        pltpu.async_copy(b_hbm.at[dstv[b]], rows_b[b], sem_b[b])
        pltpu.async_copy(c_hbm.at[pl.ds(base, _CHUNK)], rows_c[b], sem_c[b])

    def wait_rows(b):
        pltpu.make_async_copy(a_hbm.at[srcv[b]], rows_a[b], sem_a[b]).wait()
        pltpu.make_async_copy(b_hbm.at[dstv[b]], rows_b[b], sem_b[b]).wait()
        pltpu.make_async_copy(c_hbm.at[pl.ds(0, _CHUNK)], rows_c[b],
                              sem_c[b]).wait()

    fire_idx(0, 0)
    fire_idx(1, 1)
    wait_idx(0)
    fire_rows(0, 0)

    def wait_scat(b):
        pltpu.make_async_copy(msg[b], s_sh.at[dmsg[b]],
                              sem_s[b]).wait()

    def pair_body(j, carry):
        for b in range(2):
            i = 2 * j + b
            nb = 1 - b

            @pl.when(i >= 2)
            def _():
                wait_scat(b)

            pltpu.async_copy(dst_hbm.at[pl.ds(base0 + i * _CHUNK, _CHUNK)],
                             dmsg[b], sem_ds[b])

            @pl.when(i + 1 < _NCHUNK)
            def _():
                wait_idx(nb)
                fire_rows(i + 1, nb)

            wait_rows(b)

            @pl.when(i + 2 < _NCHUNK)
            def _():
                fire_idx(i + 2, b)

            @plsc.parallel_loop(0, _CHUNK, 1, unroll=2)
            def row_body(r):
                for k in range(_H // 16):
                    sl = pl.ds(k * 16, 16)
                    t = rows_a[b][r, sl] + rows_b[b][r, sl] + rows_c[b][r, sl]
                    msg[b][r, sl] = t / (1.0 + jnp.exp(-t))

            pltpu.make_async_copy(dst_hbm.at[pl.ds(0, _CHUNK)], dmsg[b],
                                  sem_ds[b]).wait()
            pltpu.async_copy(msg[b], s_sh.at[dmsg[b]], sem_s[b], add=True)
        return carry

    lax.fori_loop(0, _NCHUNK // 2, pair_body, 0)
    wait_scat(0)
    wait_scat(1)
    plsc.subcore_barrier()
    pltpu.sync_copy(s_sh.at[pl.ds(s * _RPT, _RPT)],
                    s_out.at[c, pl.ds(s * _RPT, _RPT)])


_sc_edge = functools.partial(
    pl.kernel,
    mesh=plsc.VectorSubcoreMesh(core_axis_name="c", subcore_axis_name="s",
                                num_cores=_NC, num_subcores=_NS),
    out_type=jax.ShapeDtypeStruct((_NC, _NP, _H), _f32),
    scratch_types=[
        [pltpu.VMEM((_CHUNK,), jnp.int32)] * 2,
        [pltpu.VMEM((_CHUNK,), jnp.int32)] * 2,
        [pltpu.VMEM((_CHUNK,), jnp.int32)] * 2,
        [pltpu.VMEM((_CHUNK, _H), _f32)] * 2,
        [pltpu.VMEM((_CHUNK, _H), _f32)] * 2,
        [pltpu.VMEM((_CHUNK, _H), _f32)] * 2,
        [pltpu.VMEM((_CHUNK, _H), _f32)] * 2,
        [pltpu.SemaphoreType.DMA] * 2,
        [pltpu.SemaphoreType.DMA] * 2,
        [pltpu.SemaphoreType.DMA] * 2,
        [pltpu.SemaphoreType.DMA] * 2,
        [pltpu.SemaphoreType.DMA] * 2,
        [pltpu.SemaphoreType.DMA] * 2,
        [pltpu.SemaphoreType.DMA] * 2,
        pltpu.VMEM_SHARED((_NP, _H), _f32),
    ],
)(_sc_edge_body)


def _sc_deg_body(dst_hbm, z128, ones_h, d_out, dstv, onesv, sem_id, d_sh):
    c = lax.axis_index("c")
    s = lax.axis_index("s")
    wid = s * _NC + c
    pltpu.sync_copy(z128, d_sh.at[pl.ds(s * _RPT, _RPT)])
    pltpu.sync_copy(ones_h, onesv)
    plsc.subcore_barrier()

    base0 = wid * _EPW

    def fire_idx(i, b):
        base = base0 + i * _CHUNK
        pltpu.async_copy(dst_hbm.at[pl.ds(base, _CHUNK)], dstv[b], sem_id[b])

    def wait_idx(b):
        pltpu.make_async_copy(dst_hbm.at[pl.ds(0, _CHUNK)], dstv[b],
                              sem_id[b]).wait()

    fire_idx(0, 0)
    fire_idx(1, 1)

    def pair_body(j, carry):
        for b in range(2):
            i = 2 * j + b
            wait_idx(b)
            pltpu.sync_copy(onesv, d_sh.at[dstv[b]], add=True)

            @pl.when(i + 2 < _NCHUNK)
            def _():
                fire_idx(i + 2, b)
        return carry

    lax.fori_loop(0, _NCHUNK // 2, pair_body, 0)
    plsc.subcore_barrier()
    pltpu.sync_copy(d_sh.at[pl.ds(s * _RPT, _RPT)],
                    d_out.at[c, pl.ds(s * _RPT, _RPT)])


_sc_deg = functools.partial(
    pl.kernel,
    mesh=plsc.VectorSubcoreMesh(core_axis_name="c", subcore_axis_name="s",
                                num_cores=_NC, num_subcores=_NS),
    out_type=jax.ShapeDtypeStruct((_NC, _NP, _H), _f32),
    scratch_types=[
        [pltpu.VMEM((_CHUNK,), jnp.int32)] * 2,
        pltpu.VMEM((_CHUNK, _H), _f32),
        [pltpu.SemaphoreType.DMA] * 2,
        pltpu.VMEM_SHARED((_NP, _H), _f32),
    ],
)(_sc_deg_body)


def _post_common(x, s0, s1, d0, d1, co, mw2, mb2, ux, ua, uc, ub1, uw2, ub2,
                 g, b):
    agg = (_dot(s0[...] + s1[...], mw2[...])
           + (d0[...] + d1[...]) * mb2[...])
    t = (_dot(x[...], ux[...])
         + _dot(agg, ua[...])
         + co[...] * uc[...] + ub1[...])
    h = t * jax.nn.sigmoid(t)
    y = x[...] + _dot(h, uw2[...]) + ub2[...]
    mu = jnp.mean(y, axis=-1, keepdims=True)
    yc = y - mu
    var = jnp.mean(yc * yc, axis=-1, keepdims=True)
    return yc * lax.rsqrt(var + 1e-5) * g[...] + b[...]


def _kp_body(x, s0, s1, d0, d1, co, mw2, mb2, ux, ua, uc, ub1, uw2, ub2, g, b,
             ws, wd, wcs, wcd, mb1, x_o, a_o, b_o):
    xn = _post_common(x, s0, s1, d0, d1, co, mw2, mb2, ux, ua, uc, ub1, uw2,
                      ub2, g, b)
    x_o[...] = xn
    a_o[...] = _dot(xn, ws[...]) + co[...] * wcs[...]
    b_o[...] = (_dot(xn, wd[...])
                + co[...] * wcd[...] + mb1[...])


def _kp(x, s0, s1, d0, d1, co, mw2, mb2, ux, ua, uc, ub1, uw2, ub2, g, b,
        ws, wd, wcs, wcd, mb1):
    return pl.pallas_call(
        _kp_body,
        grid=(_N // _BLK,),
        in_specs=[_rows(_H), _rows(_H), _rows(_H), _rows(1), _rows(1),
                  _rows(1), _full((_H, _H)), _full((1, _H)), _full((_H, _H)),
                  _full((_H, _H)), _full((1, _H)), _full((1, _H)),
                  _full((_H, _H)), _full((1, _H)), _full((1, _H)),
                  _full((1, _H)), _full((_H, _H)), _full((_H, _H)),
                  _full((1, _H)), _full((1, _H)), _full((1, _H))],
        out_specs=[_rows(_H), _rows(_H), _rows(_H)],
        out_shape=[jax.ShapeDtypeStruct((_N, _H), _f32)] * 3,
    )(x, s0, s1, d0, d1, co, mw2, mb2, ux, ua, uc, ub1, uw2, ub2, g, b,
      ws, wd, wcs, wcd, mb1)


def _kr_body(x, s0, s1, d0, d1, co, mw2, mb2, ux, ua, uc, ub1, uw2, ub2, g, b,
             rw1, rb1, rw2, rb2, o_o):
    xn = _post_common(x, s0, s1, d0, d1, co, mw2, mb2, ux, ua, uc, ub1, uw2,
                      ub2, g, b)
    t2 = _dot(xn, rw1[...]) + rb1[...]
    h2 = t2 * jax.nn.sigmoid(t2)
    o_o[...] = _dot(h2, rw2[...]) + rb2[...]


def _kr(x, s0, s1, d0, d1, co, mw2, mb2, ux, ua, uc, ub1, uw2, ub2, g, b,
        rw1, rb1, rw2, rb2):
    return pl.pallas_call(
        _kr_body,
        grid=(_N // _BLK,),
        in_specs=[_rows(_H), _rows(_H), _rows(_H), _rows(1), _rows(1),
                  _rows(1), _full((_H, _H)), _full((1, _H)), _full((_H, _H)),
                  _full((_H, _H)), _full((1, _H)), _full((1, _H)),
                  _full((_H, _H)), _full((1, _H)), _full((1, _H)),
                  _full((1, _H)), _full((_H, _H)), _full((1, _H)),
                  _full((_H, 1)), _full((1, 1))],
        out_specs=[_rows(1)],
        out_shape=[jax.ShapeDtypeStruct((_N, 1), _f32)],
    )(x, s0, s1, d0, d1, co, mw2, mb2, ux, ua, uc, ub1, uw2, ub2, g, b,
      rw1, rb1, rw2, rb2)[0]


def _split_layer(L):
    mw1 = L["mw1"]
    return dict(
        ws=mw1[:_H], wd=mw1[_H:2 * _H], we=mw1[2 * _H:2 * _H + _ED],
        wcs=mw1[2 * _H + _ED:2 * _H + _ED + 1],
        wcd=mw1[2 * _H + _ED + 1:2 * _H + _ED + 2],
        mb1=L["mb1"].reshape(1, _H), mw2=L["mw2"],
        mb2=L["mb2"].reshape(1, _H),
        ux=L["uw1"][:_H], ua=L["uw1"][_H:2 * _H],
        uc=L["uw1"][2 * _H:2 * _H + 1],
        ub1=L["ub1"].reshape(1, _H), uw2=L["uw2"],
        ub2=L["ub2"].reshape(1, _H),
        g=L["ln_g"].reshape(1, _H), b=L["ln_b"].reshape(1, _H),
    )


def kernel(node_features, species, edge_index, edge_features, coordination,
           params):
    p = params
    src = edge_index[0].astype(jnp.int32)
    dst = edge_index[1].astype(jnp.int32)
    sp2 = species.reshape(_N, 1).astype(jnp.int32)
    co2 = coordination.reshape(_N, 1).astype(_f32)
    emb_pad = jnp.zeros((_H, _H), _f32).at[:p["emb"].shape[0]].set(p["emb"])
    in_w = p["in_w"]
    wa, wb, wc = in_w[:_H], in_w[_H:2 * _H], in_w[2 * _H:2 * _H + 1]
    inb = p["in_b"].reshape(1, _H)
    L0 = _split_layer(p["layers"][0])
    L1 = _split_layer(p["layers"][1])

    z128 = jnp.zeros((_RPT, _H), _f32)
    ones_h = jnp.ones((_CHUNK, _H), _f32)

    dd = _sc_deg(dst, z128, ones_h)
    d0, d1 = dd[0, :_N, :1], dd[1, :_N, :1]

    x, a, b = _k0(node_features, sp2, co2, emb_pad, wa, wb, wc, inb,
                  L0["ws"], L0["wd"], L0["wcs"], L0["wcd"], L0["mb1"])
    c0 = _kc(edge_features, L0["we"])

    s = _sc_edge(a, b, c0, src, dst, z128)
    c1 = _kc(edge_features, L1["we"])
    s = s[:, :_N]
    x, a, b = _kp(x, s[0], s[1], d0, d1, co2,
                  L0["mw2"], L0["mb2"], L0["ux"], L0["ua"], L0["uc"],
                  L0["ub1"], L0["uw2"], L0["ub2"], L0["g"], L0["b"],
                  L1["ws"], L1["wd"], L1["wcs"], L1["wcd"], L1["mb1"])

    s = _sc_edge(a, b, c1, src, dst, z128)
    s = s[:, :_N]
    out = _kr(x, s[0], s[1], d0, d1, co2,
              L1["mw2"], L1["mb2"], L1["ux"], L1["ua"], L1["uc"],
              L1["ub1"], L1["uw2"], L1["ub2"], L1["g"], L1["b"],
              p["rw1"], p["rb1"].reshape(1, _H), p["rw2"],
              p["rb2"].reshape(1, 1))
    return out.reshape(_N)
